# baseline (speedup 1.0000x reference)
import functools

import jax
import jax.numpy as jnp
from jax import lax
from jax.experimental import pallas as pl
from jax.experimental.pallas import tpu as pltpu
from jax.experimental.pallas import tpu_sc as plsc

NC = 2
NS = 16
NW = NC * NS
CH = 128
LANES = 16
NBUF = 2
PK = 4
PD = 128


def _sc_scores(B, K, D):
    RPW = B // NW
    NCH = RPW // CH
    NGR = RPW // LANES
    FL = RPW * K
    NCHG = FL // CH
    GPC = CH // LANES
    mesh = plsc.VectorSubcoreMesh(
        core_axis_name="c", subcore_axis_name="s", num_cores=NC, num_subcores=NS
    )

    @functools.partial(
        pl.kernel,
        out_type=(
            jax.ShapeDtypeStruct((NW, 8, CH), jnp.float32),
            jax.ShapeDtypeStruct((NW, NCHG, CH), jnp.float32),
        ),
        mesh=mesh,
        compiler_params=pltpu.CompilerParams(needs_layout_passes=False),
        scratch_types=[
            pltpu.VMEM((NCH, CH), jnp.int32),
            pltpu.VMEM((NCH, CH), jnp.int32),
            pltpu.VMEM((NCH, CH), jnp.int32),
            pltpu.VMEM((NCH, CH), jnp.int32),
            pltpu.VMEM((NCHG, CH), jnp.int32),
            pltpu.VMEM((NCHG, CH), jnp.int32),
            pltpu.VMEM((D * NCH, CH), jnp.float32),
            pltpu.VMEM((D * NCH, CH), jnp.float32),
            pltpu.VMEM((NBUF, CH, PD), jnp.float32),
            pltpu.VMEM((8, CH), jnp.float32),
            pltpu.VMEM((NCHG, CH), jnp.float32),
            [pltpu.SemaphoreType.DMA] * NBUF,
        ],
    )
    def sc_scores(ci_h, cc_h, ni_h, nc_h, gi_h, gc_h, ine_h, oute_h,
                  pos_h, neg_h,
                  idx_c, col_c, idx_n, col_n, idx_g, col_g,
                  ct, nt, ring, pos_v, neg_v, gsems):
        wid = lax.axis_index("c") * NS + lax.axis_index("s")
        pltpu.sync_copy(ci_h.at[wid], idx_c)
        pltpu.sync_copy(cc_h.at[wid], col_c)
        pltpu.sync_copy(ni_h.at[wid], idx_n)
        pltpu.sync_copy(nc_h.at[wid], col_n)
        pltpu.sync_copy(gi_h.at[wid], idx_g)
        pltpu.sync_copy(gc_h.at[wid], col_g)

        lanes = lax.iota(jnp.int32, LANES)

        def drain(j):
            pltpu.make_async_copy(
                ine_h.at[pl.ds(0, CH)], ring.at[j], gsems[j]).wait()

        def compact(stage, colref, ch, dstref):
            def cg(g, carry):
                r16 = lanes + g * LANES
                cb16 = colref[ch, pl.ds(g * LANES, LANES)]
                for d in range(D):
                    v = plsc.load_gather(stage, [r16, cb16 + d])
                    dstref[d * NCH + ch, pl.ds(g * LANES, LANES)] = v
                return carry

            lax.fori_loop(0, GPC, cg, 0)

        cn_jobs = [(idx_c, col_c, ine_h, ct, ch) for ch in range(NCH)] + \
                  [(idx_n, col_n, oute_h, nt, ch) for ch in range(NCH)]
        for i in range(min(NBUF, len(cn_jobs))):
            idxr, _, src, _, ch = cn_jobs[i]
            pltpu.async_copy(src.at[idxr.at[ch]], ring.at[i], gsems[i])
        for i, (idxr, colr, src, dst, ch) in enumerate(cn_jobs):
            j = i % NBUF
            drain(j)
            compact(ring.at[j], colr, ch, dst)
            nxt = i + NBUF
            if nxt < len(cn_jobs):
                idxr2, _, src2, _, ch2 = cn_jobs[nxt]
                pltpu.async_copy(src2.at[idxr2.at[ch2]], ring.at[j], gsems[j])
            else:
                b = nxt - len(cn_jobs)
                if b < min(NBUF, NCHG):
                    pltpu.async_copy(
                        oute_h.at[idx_g.at[b]], ring.at[j], gsems[j])

        def pos_g(g, carry):
            sl = pl.ds((g % GPC) * LANES, LANES)
            acc = jnp.zeros((LANES,), jnp.float32)
            for d in range(D):
                acc = acc + ct[d * NCH + g // GPC, sl] * nt[d * NCH + g // GPC, sl]
            pos_v[g // GPC, pl.ds((g % GPC) * LANES, LANES)] = acc
            return carry

        lax.fori_loop(0, NGR, pos_g, 0)

        def neg_cc(cc, carry):
            for j in range(NBUF):
                c = cc * NBUF + j
                buf = ring.at[j]
                drain(j)
                base = c * CH

                def chunk_g(g, c2):
                    row16 = (base + g * LANES + lanes) // K
                    rhi = row16 // CH
                    rlo = row16 % CH
                    gc16 = col_g[c, pl.ds(g * LANES, LANES)]
                    r16 = lanes + g * LANES
                    acc = jnp.zeros((LANES,), jnp.float32)
                    for d in range(D):
                        acc = acc + (plsc.load_gather(ct, [rhi + d * NCH, rlo])
                                     * plsc.load_gather(buf, [r16, gc16 + d]))
                    neg_v[c, pl.ds(g * LANES, LANES)] = acc
                    return c2

                lax.fori_loop(0, GPC, chunk_g, 0)

                @pl.when(c + NBUF < NCHG)
                def _():
                    pltpu.async_copy(
                        oute_h.at[idx_g.at[c + NBUF]], ring.at[j], gsems[j])
            return carry

        lax.fori_loop(0, NCHG // NBUF, neg_cc, 0)

        pltpu.sync_copy(pos_v, pos_h.at[wid])
        pltpu.sync_copy(neg_v, neg_h.at[wid])

    return sc_scores


def _logsig(x):
    return jnp.minimum(x, 0.0) - jnp.log1p(jnp.exp(-jnp.abs(x)))


def _tc_loss(p_ref, n_ref, o_ref):
    lp = jnp.sum(_logsig(p_ref[...]))
    ln = jnp.sum(_logsig(-n_ref[...]))
    o_ref[0, 0] = -(lp + ln)


def kernel(centre, neighbour, neg_samples, in_emb, out_emb):
    B = centre.shape[0]
    K = neg_samples.shape[1]
    V, D = in_emb.shape
    RPW = B // NW

    VP = V // PK

    def split_idx(x, n):
        x = x.astype(jnp.int32)
        return ((x % VP).reshape(NW, n, CH),
                ((x // VP) * (PD // PK)).reshape(NW, n, CH))

    ci4, cic = split_idx(centre, RPW // CH)
    ni4, nic = split_idx(neighbour, RPW // CH)
    gi4, gic = split_idx(neg_samples, RPW * K // CH)

    def repack(t):
        return jnp.concatenate([t[i * VP:(i + 1) * VP] for i in range(PK)],
                               axis=1)

    inp = repack(in_emb)
    outp = repack(out_emb)

    pos, neg = _sc_scores(B, K, D)(ci4, cic, ni4, nic, gi4, gic, inp, outp)

    out = pl.pallas_call(
        _tc_loss,
        out_shape=jax.ShapeDtypeStruct((1, 1), jnp.float32),
        out_specs=pl.BlockSpec(memory_space=pltpu.SMEM),
    )(pos[:, :4, :].reshape(B // 128, 128),
      neg.reshape(B * K // 128, 128))
    return out[0, 0]


# speedup vs baseline: 1.0784x; 1.0784x over previous
import functools

import jax
import jax.numpy as jnp
from jax import lax
from jax.experimental import pallas as pl
from jax.experimental.pallas import tpu as pltpu
from jax.experimental.pallas import tpu_sc as plsc

NC = 2
NS = 16
NW = NC * NS
CH = 128
LANES = 16
NBUF = 2
PK = 4
PD = 128


def _sc_scores(B, K, D):
    RPW = B // NW
    NCH = RPW // CH
    NGR = RPW // LANES
    FL = RPW * K
    NCHG = FL // CH
    GPC = CH // LANES
    mesh = plsc.VectorSubcoreMesh(
        core_axis_name="c", subcore_axis_name="s", num_cores=NC, num_subcores=NS
    )

    @functools.partial(
        pl.kernel,
        out_type=(
            jax.ShapeDtypeStruct((NW, 8, CH), jnp.float32),
            jax.ShapeDtypeStruct((NW, NCHG, CH), jnp.float32),
        ),
        mesh=mesh,
        compiler_params=pltpu.CompilerParams(needs_layout_passes=False),
        scratch_types=[
            pltpu.VMEM((NCH, CH), jnp.int32),
            pltpu.VMEM((NCH, CH), jnp.int32),
            pltpu.VMEM((NCH, CH), jnp.int32),
            pltpu.VMEM((NCH, CH), jnp.int32),
            pltpu.VMEM((NCHG, CH), jnp.int32),
            pltpu.VMEM((NCHG, CH), jnp.int32),
            pltpu.VMEM((D * NCH, CH), jnp.float32),
            pltpu.VMEM((D * NCH, CH), jnp.float32),
            pltpu.VMEM((NBUF, CH, PD), jnp.float32),
            pltpu.VMEM((8, CH), jnp.float32),
            pltpu.VMEM((NCHG, CH), jnp.float32),
            [pltpu.SemaphoreType.DMA] * NBUF,
        ],
    )
    def sc_scores(ci_h, cc_h, ni_h, nc_h, gi_h, gc_h, ine_h, oute_h,
                  pos_h, neg_h,
                  idx_c, col_c, idx_n, col_n, idx_g, col_g,
                  ct, nt, ring, pos_v, neg_v, gsems):
        wid = lax.axis_index("c") * NS + lax.axis_index("s")
        pltpu.sync_copy(ci_h.at[wid], idx_c)
        pltpu.sync_copy(cc_h.at[wid], col_c)
        pltpu.sync_copy(ni_h.at[wid], idx_n)
        pltpu.sync_copy(nc_h.at[wid], col_n)
        pltpu.sync_copy(gi_h.at[wid], idx_g)
        pltpu.sync_copy(gc_h.at[wid], col_g)

        lanes = lax.iota(jnp.int32, LANES)

        def drain(j):
            pltpu.make_async_copy(
                ine_h.at[pl.ds(0, CH)], ring.at[j], gsems[j]).wait()

        def compact(stage, colref, ch, dstref):
            def cg(g, carry):
                r16 = lanes + g * LANES
                cb16 = colref[ch, pl.ds(g * LANES, LANES)]
                for d in range(D):
                    v = plsc.load_gather(stage, [r16, cb16 + d])
                    dstref[d * NCH + ch, pl.ds(g * LANES, LANES)] = v
                return carry

            lax.fori_loop(0, GPC, cg, 0)

        cn_jobs = [(idx_c, col_c, ine_h, ct, ch) for ch in range(NCH)] + \
                  [(idx_n, col_n, oute_h, nt, ch) for ch in range(NCH)]
        for i in range(min(NBUF, len(cn_jobs))):
            idxr, _, src, _, ch = cn_jobs[i]
            pltpu.async_copy(src.at[idxr.at[ch]], ring.at[i], gsems[i])
        for i, (idxr, colr, src, dst, ch) in enumerate(cn_jobs):
            j = i % NBUF
            drain(j)
            compact(ring.at[j], colr, ch, dst)
            nxt = i + NBUF
            if nxt < len(cn_jobs):
                idxr2, _, src2, _, ch2 = cn_jobs[nxt]
                pltpu.async_copy(src2.at[idxr2.at[ch2]], ring.at[j], gsems[j])
            else:
                b = nxt - len(cn_jobs)
                if b < min(NBUF, NCHG):
                    pltpu.async_copy(
                        oute_h.at[idx_g.at[b]], ring.at[j], gsems[j])

        def pos_g(g, carry):
            sl = pl.ds((g % GPC) * LANES, LANES)
            acc = jnp.zeros((LANES,), jnp.float32)
            for d in range(D):
                acc = acc + ct[d * NCH + g // GPC, sl] * nt[d * NCH + g // GPC, sl]
            pos_v[g // GPC, pl.ds((g % GPC) * LANES, LANES)] = acc
            return carry

        lax.fori_loop(0, NGR, pos_g, 0)

        def neg_cc(cc, carry):
            for j in range(NBUF):
                c = cc * NBUF + j
                buf = ring.at[j]
                drain(j)
                base = c * CH

                def chunk_g(g, c2):
                    row16 = (base + g * LANES + lanes) // K
                    rhi = row16 // CH
                    rlo = row16 % CH
                    gc16 = col_g[c, pl.ds(g * LANES, LANES)]
                    r16 = lanes + g * LANES
                    acc = jnp.zeros((LANES,), jnp.float32)
                    for d in range(D):
                        acc = acc + (plsc.load_gather(ct, [rhi + d * NCH, rlo])
                                     * plsc.load_gather(buf, [r16, gc16 + d]))
                    neg_v[c, pl.ds(g * LANES, LANES)] = acc
                    return c2

                lax.fori_loop(0, GPC, chunk_g, 0)

                @pl.when(c + NBUF < NCHG)
                def _():
                    pltpu.async_copy(
                        oute_h.at[idx_g.at[c + NBUF]], ring.at[j], gsems[j])
            return carry

        lax.fori_loop(0, NCHG // NBUF, neg_cc, 0)

        pltpu.sync_copy(pos_v, pos_h.at[wid])
        pltpu.sync_copy(neg_v, neg_h.at[wid])

    return sc_scores


def _logsig(x):
    return jnp.minimum(x, 0.0) - jnp.log1p(jnp.exp(-jnp.abs(x)))


def _tc_loss(p_ref, n_ref, o_ref):
    lp = jnp.sum(_logsig(p_ref[...]))
    ln = jnp.sum(_logsig(-n_ref[...]))
    o_ref[0, 0] = -(lp + ln)


def kernel(centre, neighbour, neg_samples, in_emb, out_emb):
    B = centre.shape[0]
    K = neg_samples.shape[1]
    V, D = in_emb.shape
    RPW = B // NW

    VP = V // PK

    def split_idx(x, n):
        x = x.astype(jnp.int32)
        return ((x % VP).reshape(NW, n, CH),
                ((x // VP) * (PD // PK)).reshape(NW, n, CH))

    ci4, cic = split_idx(centre, RPW // CH)
    ni4, nic = split_idx(neighbour, RPW // CH)
    gi4, gic = split_idx(neg_samples, RPW * K // CH)

    def _repack_body(a, b, c, d, o):
        o[:, 0:32] = a[...]
        o[:, 32:64] = b[...]
        o[:, 64:96] = c[...]
        o[:, 96:128] = d[...]

    def repack(t):
        R = 2000
        nb = VP // R
        return pl.pallas_call(
            _repack_body,
            grid=(nb,),
            in_specs=[pl.BlockSpec((R, D), lambda i, c=c: (c * nb + i, 0))
                      for c in range(PK)],
            out_specs=pl.BlockSpec((R, PD), lambda i: (i, 0)),
            out_shape=jax.ShapeDtypeStruct((VP, PD), jnp.float32),
        )(t, t, t, t)

    inp = repack(in_emb)
    outp = repack(out_emb)

    pos, neg = _sc_scores(B, K, D)(ci4, cic, ni4, nic, gi4, gic, inp, outp)

    out = pl.pallas_call(
        _tc_loss,
        out_shape=jax.ShapeDtypeStruct((1, 1), jnp.float32),
        out_specs=pl.BlockSpec(memory_space=pltpu.SMEM),
    )(pos[:, :4, :].reshape(B // 128, 128),
      neg.reshape(B * K // 128, 128))
    return out[0, 0]


# speedup vs baseline: 1.8157x; 1.6837x over previous
import functools

import jax
import jax.numpy as jnp
from jax import lax
from jax.experimental import pallas as pl
from jax.experimental.pallas import tpu as pltpu
from jax.experimental.pallas import tpu_sc as plsc

NC = 2
NS = 16
NW = NC * NS
CH = 128
LANES = 16
NBUF = 2
PK = 4
PD = 128


def _sc_scores(B, K, D):
    RPW = B // NW
    NCH = RPW // CH
    NGR = RPW // LANES
    FL = RPW * K
    NCHG = FL // CH
    GPC = CH // LANES
    mesh = plsc.VectorSubcoreMesh(
        core_axis_name="c", subcore_axis_name="s", num_cores=NC, num_subcores=NS
    )

    @functools.partial(
        pl.kernel,
        out_type=(
            jax.ShapeDtypeStruct((NW, 8, CH), jnp.float32),
            jax.ShapeDtypeStruct((NW, NCHG, CH), jnp.float32),
        ),
        mesh=mesh,
        compiler_params=pltpu.CompilerParams(needs_layout_passes=False),
        scratch_types=[
            pltpu.VMEM((NCH, CH), jnp.int32),
            pltpu.VMEM((NCH, CH), jnp.int32),
            pltpu.VMEM((NCH, CH), jnp.int32),
            pltpu.VMEM((NCH, CH), jnp.int32),
            pltpu.VMEM((NCHG, CH), jnp.int32),
            pltpu.VMEM((NCHG, CH), jnp.int32),
            pltpu.VMEM((D * NCH, CH), jnp.float32),
            pltpu.VMEM((D * NCH, CH), jnp.float32),
            pltpu.VMEM((NBUF, CH, PD), jnp.float32),
            pltpu.VMEM((8, CH), jnp.float32),
            pltpu.VMEM((NCHG, CH), jnp.float32),
            [pltpu.SemaphoreType.DMA] * NBUF,
        ],
    )
    def sc_scores(ci_h, cc_h, ni_h, nc_h, gi_h, gc_h, ine_h, oute_h,
                  pos_h, neg_h,
                  idx_c, col_c, idx_n, col_n, idx_g, col_g,
                  ct, nt, ring, pos_v, neg_v, gsems):
        wid = lax.axis_index("c") * NS + lax.axis_index("s")
        pltpu.sync_copy(ci_h.at[wid], idx_c)
        pltpu.sync_copy(cc_h.at[wid], col_c)
        pltpu.sync_copy(ni_h.at[wid], idx_n)
        pltpu.sync_copy(nc_h.at[wid], col_n)
        pltpu.sync_copy(gi_h.at[wid], idx_g)
        pltpu.sync_copy(gc_h.at[wid], col_g)

        lanes = lax.iota(jnp.int32, LANES)

        def drain(j):
            pltpu.make_async_copy(
                ine_h.at[pl.ds(0, CH)], ring.at[j], gsems[j]).wait()

        def compact(stage, colref, ch, dstref):
            def cg(g, carry):
                r16 = lanes + g * LANES
                cb16 = colref[ch, pl.ds(g * LANES, LANES)]
                for d in range(D):
                    v = plsc.load_gather(stage, [r16, cb16 + d])
                    dstref[d * NCH + ch, pl.ds(g * LANES, LANES)] = v
                return carry

            lax.fori_loop(0, GPC, cg, 0)

        cn_jobs = [(idx_c, col_c, ine_h, ct, ch) for ch in range(NCH)] + \
                  [(idx_n, col_n, oute_h, nt, ch) for ch in range(NCH)]
        for i in range(min(NBUF, len(cn_jobs))):
            idxr, _, src, _, ch = cn_jobs[i]
            pltpu.async_copy(src.at[idxr.at[ch]], ring.at[i], gsems[i])
        for i, (idxr, colr, src, dst, ch) in enumerate(cn_jobs):
            j = i % NBUF
            drain(j)
            compact(ring.at[j], colr, ch, dst)
            nxt = i + NBUF
            if nxt < len(cn_jobs):
                idxr2, _, src2, _, ch2 = cn_jobs[nxt]
                pltpu.async_copy(src2.at[idxr2.at[ch2]], ring.at[j], gsems[j])
            else:
                b = nxt - len(cn_jobs)
                if b < min(NBUF, NCHG):
                    pltpu.async_copy(
                        oute_h.at[idx_g.at[b]], ring.at[j], gsems[j])

        def pos_g(g, carry):
            sl = pl.ds((g % GPC) * LANES, LANES)
            acc = jnp.zeros((LANES,), jnp.float32)
            for d in range(D):
                acc = acc + ct[d * NCH + g // GPC, sl] * nt[d * NCH + g // GPC, sl]
            pos_v[g // GPC, pl.ds((g % GPC) * LANES, LANES)] = acc
            return carry

        lax.fori_loop(0, NGR, pos_g, 0)

        def neg_cc(cc, carry):
            for j in range(NBUF):
                c = cc * NBUF + j
                buf = ring.at[j]
                drain(j)
                base = c * CH

                def chunk_g(g, c2):
                    row16 = (base + g * LANES + lanes) // K
                    rhi = row16 // CH
                    rlo = row16 % CH
                    gc16 = col_g[c, pl.ds(g * LANES, LANES)]
                    r16 = lanes + g * LANES
                    acc = jnp.zeros((LANES,), jnp.float32)
                    for d in range(D):
                        acc = acc + (plsc.load_gather(ct, [rhi + d * NCH, rlo])
                                     * plsc.load_gather(buf, [r16, gc16 + d]))
                    neg_v[c, pl.ds(g * LANES, LANES)] = acc
                    return c2

                lax.fori_loop(0, GPC, chunk_g, 0)

                @pl.when(c + NBUF < NCHG)
                def _():
                    pltpu.async_copy(
                        oute_h.at[idx_g.at[c + NBUF]], ring.at[j], gsems[j])
            return carry

        lax.fori_loop(0, NCHG // NBUF, neg_cc, 0)

        pltpu.sync_copy(pos_v, pos_h.at[wid])
        pltpu.sync_copy(neg_v, neg_h.at[wid])

    return sc_scores


def _logsig(x):
    return jnp.minimum(x, 0.0) - jnp.log1p(jnp.exp(-jnp.abs(x)))


def _tc_loss(p_ref, n_ref, o_ref):
    lp = jnp.sum(_logsig(p_ref[...]))
    ln = jnp.sum(_logsig(-n_ref[...]))
    o_ref[0, 0] = -(lp + ln)


def kernel(centre, neighbour, neg_samples, in_emb, out_emb):
    B = centre.shape[0]
    K = neg_samples.shape[1]
    V, D = in_emb.shape
    RPW = B // NW

    CB = 8192
    RB = CB // PK
    NBLK = (V + CB - 1) // CB
    VPAD = NBLK * RB

    def split_idx(x, n):
        x = x.astype(jnp.int32)
        return (((x >> 13) * RB + (x & (RB - 1))).reshape(NW, n, CH),
                (((x >> 11) & (PK - 1)) * (PD // PK)).reshape(NW, n, CH))

    ci4, cic = split_idx(centre, RPW // CH)
    ni4, nic = split_idx(neighbour, RPW // CH)
    gi4, gic = split_idx(neg_samples, RPW * K // CH)

    def _repack_body(a, o):
        for k in range(PK):
            o[:, k * D:(k + 1) * D] = a[:, k * RB:(k + 1) * RB].T

    def repack(t):
        return pl.pallas_call(
            _repack_body,
            grid=(NBLK,),
            in_specs=[pl.BlockSpec((D, CB), lambda i: (0, i))],
            out_specs=pl.BlockSpec((RB, PD), lambda i: (i, 0)),
            out_shape=jax.ShapeDtypeStruct((VPAD, PD), jnp.float32),
        )(t.T)

    inp = repack(in_emb)
    outp = repack(out_emb)

    pos, neg = _sc_scores(B, K, D)(ci4, cic, ni4, nic, gi4, gic, inp, outp)

    out = pl.pallas_call(
        _tc_loss,
        out_shape=jax.ShapeDtypeStruct((1, 1), jnp.float32),
        out_specs=pl.BlockSpec(memory_space=pltpu.SMEM),
    )(pos[:, :4, :].reshape(B // 128, 128),
      neg.reshape(B * K // 128, 128))
    return out[0, 0]


# speedup vs baseline: 1.8334x; 1.0098x over previous
import functools

import jax
import jax.numpy as jnp
from jax import lax
from jax.experimental import pallas as pl
from jax.experimental.pallas import tpu as pltpu
from jax.experimental.pallas import tpu_sc as plsc

NC = 2
NS = 16
NW = NC * NS
CH = 128
LANES = 16
NBUF = 2
PK = 4
PD = 128


def _sc_scores(B, K, D):
    RPW = B // NW
    NCH = RPW // CH
    NGR = RPW // LANES
    FL = RPW * K
    NCHG = FL // CH
    GPC = CH // LANES
    mesh = plsc.VectorSubcoreMesh(
        core_axis_name="c", subcore_axis_name="s", num_cores=NC, num_subcores=NS
    )

    @functools.partial(
        pl.kernel,
        out_type=(
            jax.ShapeDtypeStruct((NW, 8, CH), jnp.float32),
            jax.ShapeDtypeStruct((NW, NCHG, CH), jnp.float32),
        ),
        mesh=mesh,
        compiler_params=pltpu.CompilerParams(needs_layout_passes=False),
        scratch_types=[
            pltpu.VMEM((NCH, CH), jnp.int32),
            pltpu.VMEM((NCH, CH), jnp.int32),
            pltpu.VMEM((NCH, CH), jnp.int32),
            pltpu.VMEM((NCH, CH), jnp.int32),
            pltpu.VMEM((NCHG, CH), jnp.int32),
            pltpu.VMEM((NCHG, CH), jnp.int32),
            pltpu.VMEM((D * NCH, CH), jnp.float32),
            pltpu.VMEM((D * NCH, CH), jnp.float32),
            pltpu.VMEM((NBUF, CH, PD), jnp.float32),
            pltpu.VMEM((8, CH), jnp.float32),
            pltpu.VMEM((NCHG, CH), jnp.float32),
            [pltpu.SemaphoreType.DMA] * NBUF,
        ],
    )
    def sc_scores(ci_h, cc_h, ni_h, nc_h, gi_h, gc_h, ine_h, oute_h,
                  pos_h, neg_h,
                  idx_c, col_c, idx_n, col_n, idx_g, col_g,
                  ct, nt, ring, pos_v, neg_v, gsems):
        wid = lax.axis_index("c") * NS + lax.axis_index("s")
        pltpu.sync_copy(ci_h.at[wid], idx_c)
        pltpu.sync_copy(cc_h.at[wid], col_c)
        pltpu.sync_copy(ni_h.at[wid], idx_n)
        pltpu.sync_copy(nc_h.at[wid], col_n)
        pltpu.sync_copy(gi_h.at[wid], idx_g)
        pltpu.sync_copy(gc_h.at[wid], col_g)

        lanes = lax.iota(jnp.int32, LANES)

        def drain(j):
            pltpu.make_async_copy(
                ine_h.at[pl.ds(0, CH)], ring.at[j], gsems[j]).wait()

        def compact(stage, colref, ch, dstref):
            def cg(g, carry):
                r16 = lanes + g * LANES
                cb16 = colref[ch, pl.ds(g * LANES, LANES)]
                for d in range(D):
                    v = plsc.load_gather(stage, [r16, cb16 + d])
                    dstref[d * NCH + ch, pl.ds(g * LANES, LANES)] = v
                return carry

            lax.fori_loop(0, GPC, cg, 0)

        cn_jobs = [(idx_c, col_c, ine_h, ct, ch) for ch in range(NCH)] + \
                  [(idx_n, col_n, oute_h, nt, ch) for ch in range(NCH)]
        for i in range(min(NBUF, len(cn_jobs))):
            idxr, _, src, _, ch = cn_jobs[i]
            pltpu.async_copy(src.at[idxr.at[ch]], ring.at[i], gsems[i])
        for i, (idxr, colr, src, dst, ch) in enumerate(cn_jobs):
            j = i % NBUF
            drain(j)
            compact(ring.at[j], colr, ch, dst)
            nxt = i + NBUF
            if nxt < len(cn_jobs):
                idxr2, _, src2, _, ch2 = cn_jobs[nxt]
                pltpu.async_copy(src2.at[idxr2.at[ch2]], ring.at[j], gsems[j])
            else:
                b = nxt - len(cn_jobs)
                if b < min(NBUF, NCHG):
                    pltpu.async_copy(
                        oute_h.at[idx_g.at[b]], ring.at[j], gsems[j])

        def pos_g(g, carry):
            sl = pl.ds((g % GPC) * LANES, LANES)
            acc = jnp.zeros((LANES,), jnp.float32)
            for d in range(D):
                acc = acc + ct[d * NCH + g // GPC, sl] * nt[d * NCH + g // GPC, sl]
            pos_v[g // GPC, pl.ds((g % GPC) * LANES, LANES)] = acc
            return carry

        lax.fori_loop(0, NGR, pos_g, 0)

        def neg_cc(cc, carry):
            for j in range(NBUF):
                c = cc * NBUF + j
                buf = ring.at[j]
                drain(j)
                base = c * CH

                def chunk_g(g, c2):
                    row16 = (base + g * LANES + lanes) // K
                    rhi = row16 // CH
                    rlo = row16 % CH
                    gc16 = col_g[c, pl.ds(g * LANES, LANES)]
                    r16 = lanes + g * LANES
                    acc = jnp.zeros((LANES,), jnp.float32)
                    for d in range(D):
                        acc = acc + (plsc.load_gather(ct, [rhi + d * NCH, rlo])
                                     * plsc.load_gather(buf, [r16, gc16 + d]))
                    neg_v[c, pl.ds(g * LANES, LANES)] = acc
                    return c2

                lax.fori_loop(0, GPC, chunk_g, 0)

                @pl.when(c + NBUF < NCHG)
                def _():
                    pltpu.async_copy(
                        oute_h.at[idx_g.at[c + NBUF]], ring.at[j], gsems[j])
            return carry

        lax.fori_loop(0, NCHG // NBUF, neg_cc, 0)

        pltpu.sync_copy(pos_v, pos_h.at[wid])
        pltpu.sync_copy(neg_v, neg_h.at[wid])

    return sc_scores


def _logsig(x):
    return jnp.minimum(x, 0.0) - jnp.log1p(jnp.exp(-jnp.abs(x)))


def _tc_loss(p_ref, n_ref, o_ref):
    lp = jnp.sum(_logsig(p_ref[...]))
    ln = jnp.sum(_logsig(-n_ref[...]))
    o_ref[0, 0] = -(lp + ln)


def kernel(centre, neighbour, neg_samples, in_emb, out_emb):
    B = centre.shape[0]
    K = neg_samples.shape[1]
    V, D = in_emb.shape
    RPW = B // NW

    CB = 16384
    RB = CB // PK
    SB = CB.bit_length() - 1
    SR = RB.bit_length() - 1
    NBLK = (V + CB - 1) // CB
    VPAD = NBLK * RB

    def split_idx(x, n):
        x = x.astype(jnp.int32)
        return (((x >> SB) * RB + (x & (RB - 1))).reshape(NW, n, CH),
                (((x >> SR) & (PK - 1)) * (PD // PK)).reshape(NW, n, CH))

    ci4, cic = split_idx(centre, RPW // CH)
    ni4, nic = split_idx(neighbour, RPW // CH)
    gi4, gic = split_idx(neg_samples, RPW * K // CH)

    def _repack_body(a, o):
        for k in range(PK):
            o[:, k * D:(k + 1) * D] = a[:, k * RB:(k + 1) * RB].T

    def repack(t):
        return pl.pallas_call(
            _repack_body,
            grid=(NBLK,),
            in_specs=[pl.BlockSpec((D, CB), lambda i: (0, i))],
            out_specs=pl.BlockSpec((RB, PD), lambda i: (i, 0)),
            out_shape=jax.ShapeDtypeStruct((VPAD, PD), jnp.float32),
        )(t.T)

    inp = repack(in_emb)
    outp = repack(out_emb)

    pos, neg = _sc_scores(B, K, D)(ci4, cic, ni4, nic, gi4, gic, inp, outp)

    out = pl.pallas_call(
        _tc_loss,
        out_shape=jax.ShapeDtypeStruct((1, 1), jnp.float32),
        out_specs=pl.BlockSpec(memory_space=pltpu.SMEM),
    )(pos[:, :4, :].reshape(B // 128, 128),
      neg.reshape(B * K // 128, 128))
    return out[0, 0]
